# baseline (speedup 1.0000x reference)
import functools

import jax
import jax.numpy as jnp
from jax import lax
from jax.experimental import pallas as pl
from jax.experimental.pallas import tpu as pltpu
from jax.experimental.pallas import tpu_sc as plsc

ENT_SIZE = 1000000
DIM = 32
NUM_CHUNK = 16
POS_NUM = 1024
NEG_NUM = 1024
PN = POS_NUM + NEG_NUM

_IDX_ROWS = 4 * NUM_CHUNK * POS_NUM // 128
_NW = 32
_ROWS_PER_W = _IDX_ROWS // _NW


@functools.cache
def _make_sc_gather():
    @functools.partial(
        pl.kernel,
        mesh=plsc.VectorSubcoreMesh(core_axis_name="c", subcore_axis_name="s"),
        out_type=jax.ShapeDtypeStruct((_IDX_ROWS, 128, 128), jnp.float32),
        scratch_types=[
            pltpu.VMEM((_ROWS_PER_W, 128), jnp.int32),
            pltpu.VMEM((2, 128, 128), jnp.float32),
            pltpu.SemaphoreType.DMA,
            pltpu.SemaphoreType.DMA,
        ],
    )
    def _sc_gather(emb_hbm, idx_hbm, out_hbm, idx_v, rows_v, sem0, sem1):
        wid = lax.axis_index("s") * 2 + lax.axis_index("c")
        base = wid * _ROWS_PER_W
        pltpu.sync_copy(idx_hbm.at[pl.ds(base, _ROWS_PER_W)], idx_v)
        sems = (sem0, sem1)
        copies = [None, None]
        copies[0] = pltpu.async_copy(emb_hbm.at[idx_v.at[0]], rows_v.at[0], sem0)
        for j in range(_ROWS_PER_W):
            if j + 1 < _ROWS_PER_W:
                copies[(j + 1) % 2] = pltpu.async_copy(
                    emb_hbm.at[idx_v.at[j + 1]], rows_v.at[(j + 1) % 2],
                    sems[(j + 1) % 2],
                )
            copies[j % 2].wait()
            pltpu.sync_copy(rows_v.at[j % 2], out_hbm.at[base + j])

    return _sc_gather


def _tc_score_body(g_ref, q_ref, pos_ref, hn_ref, tn_ref):
    def sel(t):
        g = g_ref[t, 0]
        qv = q_ref[t, 0]
        out = g[:, 0:DIM]
        for qq in (1, 2, 3):
            out = jnp.where(qv == qq, g[:, qq * DIM:(qq + 1) * DIM], out)
        return out

    head = sel(0)
    tail = sel(1)
    hne = sel(2)
    tne = sel(3)
    dn = (((1,), (1,)), ((), ()))
    s_ht = lax.dot_general(head, tail, dn, preferred_element_type=jnp.float32)
    s_hn = lax.dot_general(head, tne, dn, preferred_element_type=jnp.float32)
    s_th = lax.dot_general(tail, head, dn, preferred_element_type=jnp.float32)
    s_tn = lax.dot_general(tail, hne, dn, preferred_element_type=jnp.float32)
    rows = lax.broadcasted_iota(jnp.int32, (POS_NUM, POS_NUM), 0)
    cols = lax.broadcasted_iota(jnp.int32, (POS_NUM, POS_NUM), 1)
    neg = jnp.where(rows == cols, jnp.float32(-1000000000.0), jnp.float32(0.0))
    hn_ref[0, :, 0:POS_NUM] = s_ht + neg
    hn_ref[0, :, POS_NUM:PN] = s_hn
    tn_ref[0, :, 0:POS_NUM] = s_th + neg
    tn_ref[0, :, POS_NUM:PN] = s_tn
    pos_ref[0] = jnp.sum(head * tail, axis=1, keepdims=True)


_tc_score = pl.pallas_call(
    _tc_score_body,
    grid=(NUM_CHUNK,),
    in_specs=[
        pl.BlockSpec((4, 1, POS_NUM, 128), lambda c: (0, c, 0, 0)),
        pl.BlockSpec((4, 1, POS_NUM, 1), lambda c: (0, c, 0, 0)),
    ],
    out_specs=[
        pl.BlockSpec((1, POS_NUM, 1), lambda c: (c, 0, 0)),
        pl.BlockSpec((1, POS_NUM, PN), lambda c: (c, 0, 0)),
        pl.BlockSpec((1, POS_NUM, PN), lambda c: (c, 0, 0)),
    ],
    out_shape=[
        jax.ShapeDtypeStruct((NUM_CHUNK, POS_NUM, 1), jnp.float32),
        jax.ShapeDtypeStruct((NUM_CHUNK, POS_NUM, PN), jnp.float32),
        jax.ShapeDtypeStruct((NUM_CHUNK, POS_NUM, PN), jnp.float32),
    ],
)


def kernel(head_index, tail_index, head_neg_index, tail_neg_index, rel_index, emb):
    del rel_index
    idx = jnp.stack(
        [head_index, tail_index, head_neg_index, tail_neg_index]
    ).astype(jnp.int32)
    idx_grid = (idx >> 2).reshape(_IDX_ROWS, 128)
    q = (idx & 3).reshape(4, NUM_CHUNK, POS_NUM, 1)
    emb128 = emb.reshape(ENT_SIZE // 4, 128)
    gathered = _make_sc_gather()(emb128, idx_grid)
    g = gathered.reshape(4, NUM_CHUNK, POS_NUM, 128)
    pos, hn, tn = _tc_score(g, q)
    pos2 = pos.reshape(NUM_CHUNK * POS_NUM, 1)
    return (
        pos2,
        pos2,
        hn.reshape(NUM_CHUNK * POS_NUM, PN),
        tn.reshape(NUM_CHUNK * POS_NUM, PN),
    )


# speedup vs baseline: 1.5541x; 1.5541x over previous
import functools

import jax
import jax.numpy as jnp
from jax import lax
from jax.experimental import pallas as pl
from jax.experimental.pallas import tpu as pltpu
from jax.experimental.pallas import tpu_sc as plsc

ENT_SIZE = 1000000
DIM = 32
NUM_CHUNK = 16
POS_NUM = 1024
NEG_NUM = 1024
PN = POS_NUM + NEG_NUM
PACK = 4
_T_BLK = 8192
_SUB = _T_BLK // PACK
_N_T_BLK = (ENT_SIZE + _T_BLK - 1) // _T_BLK
N_PACKED = _N_T_BLK * _SUB

_IDX_ROWS = 4 * NUM_CHUNK * POS_NUM // 128
_NW = 32
_ROWS_PER_W = _IDX_ROWS // _NW


def _tc_repack_body(t_ref, out_ref):
    x = t_ref[...]
    parts = [
        jnp.transpose(x[:, a * _SUB:(a + 1) * _SUB], (1, 0))
        for a in range(PACK)
    ]
    out_ref[...] = jnp.concatenate(parts, axis=1)


_tc_repack = pl.pallas_call(
    _tc_repack_body,
    grid=(_N_T_BLK,),
    in_specs=[pl.BlockSpec((DIM, _T_BLK), lambda i: (0, i))],
    out_specs=pl.BlockSpec((_SUB, PACK * DIM), lambda i: (i, 0)),
    out_shape=jax.ShapeDtypeStruct((N_PACKED, PACK * DIM), jnp.float32),
)


@functools.cache
def _make_sc_gather():
    @functools.partial(
        pl.kernel,
        mesh=plsc.VectorSubcoreMesh(core_axis_name="c", subcore_axis_name="s"),
        out_type=jax.ShapeDtypeStruct((_IDX_ROWS, 128, 128), jnp.float32),
        scratch_types=[
            pltpu.VMEM((_ROWS_PER_W, 128), jnp.int32),
            pltpu.VMEM((2, 128, 128), jnp.float32),
            pltpu.SemaphoreType.DMA,
            pltpu.SemaphoreType.DMA,
        ],
    )
    def _sc_gather(tab_hbm, idx_hbm, out_hbm, idx_v, rows_v, sem0, sem1):
        wid = lax.axis_index("s") * 2 + lax.axis_index("c")
        base = wid * _ROWS_PER_W
        pltpu.sync_copy(idx_hbm.at[pl.ds(base, _ROWS_PER_W)], idx_v)
        sems = (sem0, sem1)
        copies = [None, None]
        copies[0] = pltpu.async_copy(tab_hbm.at[idx_v.at[0]], rows_v.at[0], sem0)
        for j in range(_ROWS_PER_W):
            b = j % 2
            if j + 1 < _ROWS_PER_W:
                copies[(j + 1) % 2] = pltpu.async_copy(
                    tab_hbm.at[idx_v.at[j + 1]], rows_v.at[(j + 1) % 2],
                    sems[(j + 1) % 2],
                )
            copies[b].wait()
            pltpu.sync_copy(rows_v.at[b], out_hbm.at[base + j])

    return _sc_gather


def _tc_score_body(g_ref, q_ref, pos_ref, hn_ref, tn_ref):
    def sel(t):
        g = g_ref[t, 0]
        qv = q_ref[t, 0]
        out = g[:, 0:DIM]
        for qq in range(1, PACK):
            out = jnp.where(qv == qq, g[:, qq * DIM:(qq + 1) * DIM], out)
        return out

    head = sel(0)
    tail = sel(1)
    hne = sel(2)
    tne = sel(3)
    dn = (((1,), (1,)), ((), ()))
    s_ht = lax.dot_general(head, tail, dn, preferred_element_type=jnp.float32)
    s_hn = lax.dot_general(head, tne, dn, preferred_element_type=jnp.float32)
    s_th = lax.dot_general(tail, head, dn, preferred_element_type=jnp.float32)
    s_tn = lax.dot_general(tail, hne, dn, preferred_element_type=jnp.float32)
    rows = lax.broadcasted_iota(jnp.int32, (POS_NUM, POS_NUM), 0)
    cols = lax.broadcasted_iota(jnp.int32, (POS_NUM, POS_NUM), 1)
    neg = jnp.where(rows == cols, jnp.float32(-1000000000.0), jnp.float32(0.0))
    hn_ref[0, :, 0:POS_NUM] = s_ht + neg
    hn_ref[0, :, POS_NUM:PN] = s_hn
    tn_ref[0, :, 0:POS_NUM] = s_th + neg
    tn_ref[0, :, POS_NUM:PN] = s_tn
    pos_ref[0] = jnp.sum(head * tail, axis=1, keepdims=True)


_tc_score = pl.pallas_call(
    _tc_score_body,
    grid=(NUM_CHUNK,),
    in_specs=[
        pl.BlockSpec((4, 1, POS_NUM, 128), lambda c: (0, c, 0, 0)),
        pl.BlockSpec((4, 1, POS_NUM, 1), lambda c: (0, c, 0, 0)),
    ],
    out_specs=[
        pl.BlockSpec((1, POS_NUM, 1), lambda c: (c, 0, 0)),
        pl.BlockSpec((1, POS_NUM, PN), lambda c: (c, 0, 0)),
        pl.BlockSpec((1, POS_NUM, PN), lambda c: (c, 0, 0)),
    ],
    out_shape=[
        jax.ShapeDtypeStruct((NUM_CHUNK, POS_NUM, 1), jnp.float32),
        jax.ShapeDtypeStruct((NUM_CHUNK, POS_NUM, PN), jnp.float32),
        jax.ShapeDtypeStruct((NUM_CHUNK, POS_NUM, PN), jnp.float32),
    ],
)


def kernel(head_index, tail_index, head_neg_index, tail_neg_index, rel_index, emb):
    del rel_index
    idx = jnp.stack(
        [head_index, tail_index, head_neg_index, tail_neg_index]
    ).astype(jnp.int32)
    row = (idx >> 13) * _SUB + (idx & (_SUB - 1))
    idx_grid = row.reshape(_IDX_ROWS, 128)
    q = ((idx >> 11) & 3).reshape(4, NUM_CHUNK, POS_NUM, 1)
    emb_t = emb.T
    tab = _tc_repack(emb_t)
    gathered = _make_sc_gather()(tab, idx_grid)
    g = gathered.reshape(4, NUM_CHUNK, POS_NUM, 128)
    pos, hn, tn = _tc_score(g, q)
    pos2 = pos.reshape(NUM_CHUNK * POS_NUM, 1)
    return (
        pos2,
        pos2,
        hn.reshape(NUM_CHUNK * POS_NUM, PN),
        tn.reshape(NUM_CHUNK * POS_NUM, PN),
    )
